# baseline (speedup 1.0000x reference)
import functools

import jax
import jax.numpy as jnp
from jax import lax
from jax.experimental import pallas as pl
from jax.experimental.pallas import tpu as pltpu
from jax.experimental.pallas import tpu_sc as plsc

B = 2
C = 256
K = 184
H = 224
W = 224

NW = 32
RB = 8
NPOS = B * (H // RB)
PPB = H // RB
KC = 23
NKC = K // KC
NG = W // 16

P = H * W
NPX = 3584
NBLK = P // NPX


def _sc_argmax_body(bb, mask_hbm, idx_hbm, buf0, buf1, runmax, runidx, sem0, sem1):
    wid = lax.axis_index("s") * 2 + lax.axis_index("c")
    bufs = (buf0, buf1)
    sems = (sem0, sem1)

    def src(i):
        q = i
        h0 = pl.multiple_of((wid % PPB) * RB, RB)
        return wid, mask_hbm.at[bb, pl.ds(q * KC, KC), pl.ds(h0, RB), :]

    def compute(i):
        q = i
        buf = bufs[i % 2]
        qbase = q * KC
        sls = [pl.ds(g * 16, 16) for g in range(NG)]

        def row_body(r, _):
            if q == 0:
                init = tuple(
                    (buf[0, r, sl], jnp.zeros((16,), jnp.int32)) for sl in sls
                )
                lo = 1
            else:
                init = tuple((runmax[r, sl], runidx[r, sl]) for sl in sls)
                lo = 0

            def ch_body(ch, carry):
                chv = jnp.full((16,), qbase + ch, jnp.int32)
                out = []
                for g, sl in enumerate(sls):
                    maxv, idxv = carry[g]
                    v = buf[ch, r, sl]
                    pred = v > maxv
                    out.append((jnp.where(pred, v, maxv),
                                jnp.where(pred, chv, idxv)))
                return tuple(out)

            res = lax.fori_loop(lo, KC, ch_body, init)
            for g, sl in enumerate(sls):
                if q < NKC - 1:
                    runmax[r, sl] = res[g][0]
                runidx[r, sl] = res[g][1]
            return 0

        lax.fori_loop(0, RB, row_body, 0)

        if q == NKC - 1:
            h0 = pl.multiple_of((wid % PPB) * RB, RB)
            pltpu.sync_copy(runidx, idx_hbm.at[0, pl.ds(h0, RB), :])

    NT = NKC

    def valid(i):
        return wid < PPB

    _, s0 = src(0)

    @pl.when(valid(0))
    def _():
        pltpu.make_async_copy(s0, bufs[0], sems[0]).start()
    for i in range(NT):
        if i + 1 < NT:
            _, sn = src(i + 1)

            @pl.when(valid(i + 1))
            def _(sn=sn, i=i):
                pltpu.make_async_copy(sn, bufs[(i + 1) % 2], sems[(i + 1) % 2]).start()

        _, s = src(i)

        @pl.when(valid(i))
        def _(s=s, i=i):
            pltpu.make_async_copy(s, bufs[i % 2], sems[i % 2]).wait()
            compute(i)


def _sc_argmax(mask, bb):
    return pl.kernel(
        functools.partial(_sc_argmax_body, bb),
        out_type=jax.ShapeDtypeStruct((1, H, W), jnp.int32),
        mesh=plsc.VectorSubcoreMesh(core_axis_name="c", subcore_axis_name="s"),
        scratch_types=[
            pltpu.VMEM((KC, RB, W), jnp.float32),
            pltpu.VMEM((KC, RB, W), jnp.float32),
            pltpu.VMEM((RB, W), jnp.float32),
            pltpu.VMEM((RB, W), jnp.int32),
            pltpu.SemaphoreType.DMA,
            pltpu.SemaphoreType.DMA,
        ],
    )(mask)


def _tc_affine_body(wb_ref, idx_ref, in_ref, out_ref):
    idxc = idx_ref[0, 0, :].reshape(NPX, 1)
    iot = lax.broadcasted_iota(jnp.int32, (NPX, K), 1)
    onehot = (iot == idxc).astype(jnp.bfloat16)
    g = jnp.dot(onehot, wb_ref[...], preferred_element_type=jnp.float32)
    out_ref[0] = in_ref[0] * g[:, :C] + g[:, C:]


def _tc_affine_body_alias(wb_ref, idx_ref, in_ref, carry_ref, out_ref):
    del carry_ref
    _tc_affine_body(wb_ref, idx_ref, in_ref, out_ref)


def _tc_affine(wb, idxf3, inp2, bb, carry=None):
    in_specs = [
        pl.BlockSpec((K, 2 * C), lambda b, j: (0, 0)),
        pl.BlockSpec((1, 1, NPX), lambda b, j: (b, 0, j)),
        pl.BlockSpec((1, NPX, C), lambda b, j, bb=bb: (bb, j, 0)),
    ]
    args = [wb, idxf3, inp2]
    kwargs = {}
    body = _tc_affine_body
    if carry is not None:
        in_specs.append(pl.BlockSpec(memory_space=pl.ANY))
        args.append(carry)
        kwargs["input_output_aliases"] = {3: 0}
        body = _tc_affine_body_alias
    return pl.pallas_call(
        body,
        grid=(1, NBLK),
        in_specs=in_specs,
        out_specs=pl.BlockSpec((1, NPX, C), lambda b, j, bb=bb: (bb, j, 0)),
        out_shape=jax.ShapeDtypeStruct((B, P, C), jnp.float32),
        **kwargs,
    )(*args)


def kernel(input, mask, weight, bias):
    wb = jnp.concatenate([weight, bias], axis=1).astype(jnp.bfloat16)
    inp2 = input.transpose(0, 2, 3, 1).reshape(B, P, C)
    idx0 = _sc_argmax(mask, 0)
    out2 = _tc_affine(wb, idx0.reshape(1, 1, P), inp2, 0)
    idx1 = _sc_argmax(mask, 1)
    out2 = _tc_affine(wb, idx1.reshape(1, 1, P), inp2, 1, carry=out2)
    return out2.reshape(B, H, W, C).transpose(0, 3, 1, 2)
